# Initial kernel scaffold; baseline (speedup 1.0000x reference)
import jax
import jax.numpy as jnp
from jax.experimental import pallas as pl


def kernel(W0, b0, W1):
    raise NotImplementedError("write your pallas kernel here")


# speedup vs baseline: 23.6633x; 23.6633x over previous
import numpy as np
import jax
import jax.numpy as jnp
from jax.experimental import pallas as pl

_W = 24
_NH = 32
_S = 2 * _W - 1
_STRIP = _S * _W
_STRIP_P = 1152
_NCOLS = _W * _STRIP_P


def _logcoord(x):
    x = x.astype(np.float64) / float(_W - 1) * 8.0
    return (np.sign(x) * np.log2(np.abs(x) + 1.0) / np.log2(8.0)).astype(np.float32)


def _build_cs_t():
    iw = np.arange(_W)[:, None, None]
    s = np.arange(_S)[None, :, None]
    t = np.arange(_W)[None, None, :]
    ch = _logcoord(np.broadcast_to(_W - 1 - s, (_W, _S, _W)))
    cw = _logcoord(np.broadcast_to(iw - t, (_W, _S, _W)))
    cs = np.zeros((_W, _STRIP_P, 2), np.float32)
    cs[:, :_STRIP, 0] = ch.reshape(_W, _STRIP)
    cs[:, :_STRIP, 1] = cw.reshape(_W, _STRIP)
    return np.ascontiguousarray(cs.reshape(_NCOLS, 2).T)


_CS_T_NP = _build_cs_t()

_MLP_GRID = 8
_NB = _NCOLS // _MLP_GRID


def _mlp_body(cs_ref, w0t_ref, b0_ref, w1t_ref, out_ref):
    c0 = cs_ref[0:1, :]
    c1 = cs_ref[1:2, :]
    a0 = w0t_ref[:, 0:1]
    a1 = w0t_ref[:, 1:2]
    h = jnp.maximum(a0 * c0 + a1 * c1 + b0_ref[:, :], 0.0)
    z = jnp.dot(w1t_ref[:, :], h, preferred_element_type=jnp.float32)
    out_ref[:, :] = 16.0 * jax.nn.sigmoid(z)


def _expand_body(strip_ref, out_ref):
    for ih in range(_W):
        lo = (_W - 1 - ih) * _W
        out_ref[0, ih * _W:(ih + 1) * _W, :] = strip_ref[0, :, lo:lo + _W * _W]


def kernel(W0, b0, W1):
    cs_t = jnp.asarray(_CS_T_NP)
    strip = pl.pallas_call(
        _mlp_body,
        grid=(_MLP_GRID,),
        in_specs=[
            pl.BlockSpec((2, _NB), lambda i: (0, i)),
            pl.BlockSpec((512, 2), lambda i: (0, 0)),
            pl.BlockSpec((512, 1), lambda i: (0, 0)),
            pl.BlockSpec((_NH, 512), lambda i: (0, 0)),
        ],
        out_specs=pl.BlockSpec((_NH, _NB), lambda i: (0, i)),
        out_shape=jax.ShapeDtypeStruct((_NH, _NCOLS), jnp.float32),
    )(cs_t, W0.T, b0.reshape(512, 1), W1.T)

    strip = strip.reshape(_NH, _W, _STRIP_P)
    out = pl.pallas_call(
        _expand_body,
        grid=(_NH,),
        in_specs=[pl.BlockSpec((1, _W, _STRIP_P), lambda h: (h, 0, 0))],
        out_specs=pl.BlockSpec((1, _W * _W, _W * _W), lambda h: (h, 0, 0)),
        out_shape=jax.ShapeDtypeStruct((_NH, _W * _W, _W * _W), jnp.float32),
    )(strip)
    return out
